# baseline (speedup 1.0000x reference)
import functools

import jax
import jax.numpy as jnp
from jax import lax
from jax.experimental import pallas as pl
from jax.experimental.pallas import tpu as pltpu
from jax.experimental.pallas import tpu_sc as plsc

N_FACT = 32
N_ROWS = 1000000
B = 4096
NC = 2
NS = 16
NW = NC * NS
BPW = B // NW
L = 16
GROUPS = BPW // L
CHUNK = 1
NCH = BPW // CHUNK
TILE_W = 128
DEPTH = 12
BIAS_ROWS = 7816
BIAS_PAD = BIAS_ROWS * TILE_W - N_ROWS


def _main_body(users_h, items_h, uet_h, iet_h, out_h,
               idx_u, idx_i, ue_slabs, ie_slabs, out_v,
               sem0, sem1, sem2, sem3, sem4, sem5, sem6, sem7, sem8, sem9,
               sem10, sem11):
    wid = lax.axis_index("s") * NC + lax.axis_index("c")
    base = wid * BPW

    pltpu.sync_copy(users_h.at[pl.ds(base, BPW)], idx_u)
    pltpu.sync_copy(items_h.at[pl.ds(base, BPW)], idx_i)

    vu = [idx_u[pl.ds(g * L, L)] for g in range(GROUPS)]
    vi = [idx_i[pl.ds(g * L, L)] for g in range(GROUPS)]

    lane = lax.broadcasted_iota(jnp.int32, (L,), 0)
    lane_hi = lane + L
    lane_masks = [lane == j for j in range(L)]
    sems = [sem0, sem1, sem2, sem3, sem4, sem5, sem6, sem7, sem8, sem9,
            sem10, sem11]

    def issue(k):
        par = k % DEPTH
        cps = []
        for s in range(CHUNK):
            b = k * CHUNK + s
            g, j = b // L, b % L
            cu = vu[g][j]
            ci = vi[g][j]
            cu0 = pl.multiple_of((cu >> 7) << 7, TILE_W)
            ci0 = pl.multiple_of((ci >> 7) << 7, TILE_W)
            cps.append(pltpu.async_copy(
                uet_h.at[:, pl.ds(cu0, TILE_W)], ue_slabs.at[par, s],
                sems[par]))
            cps.append(pltpu.async_copy(
                iet_h.at[:, pl.ds(ci0, TILE_W)], ie_slabs.at[par, s],
                sems[par]))
        return cps

    acc = jnp.zeros((L,), jnp.float32)
    inflight = [issue(k) for k in range(DEPTH - 1)]
    for k in range(NCH):
        if k + DEPTH - 1 < NCH:
            inflight.append(issue(k + DEPTH - 1))
        for cp in inflight.pop(0):
            cp.wait()
        par = k % DEPTH
        for s in range(CHUNK):
            b = k * CHUNK + s
            g, j = b // L, b % L
            ju = jnp.full((L,), vu[g][j] & (TILE_W - 1), jnp.int32)
            ji = jnp.full((L,), vi[g][j] & (TILE_W - 1), jnp.int32)
            u0 = plsc.load_gather(ue_slabs.at[par, s], [lane, ju])
            u1 = plsc.load_gather(ue_slabs.at[par, s], [lane_hi, ju])
            i0 = plsc.load_gather(ie_slabs.at[par, s], [lane, ji])
            i1 = plsc.load_gather(ie_slabs.at[par, s], [lane_hi, ji])
            dot = jnp.sum(u0 * i0 + u1 * i1)
            acc = jnp.where(lane_masks[j], jnp.full((L,), dot), acc)
            if j == L - 1:
                out_v[pl.ds(g * L, L)] = acc

    pltpu.sync_copy(out_v, out_h.at[pl.ds(base, BPW)])


_mf_main = pl.kernel(
    _main_body,
    out_type=jax.ShapeDtypeStruct((B,), jnp.float32),
    mesh=plsc.VectorSubcoreMesh(core_axis_name="c", subcore_axis_name="s"),
    compiler_params=pltpu.CompilerParams(needs_layout_passes=False),
    scratch_types=[
        pltpu.VMEM((BPW,), jnp.int32),
        pltpu.VMEM((BPW,), jnp.int32),
        pltpu.VMEM((DEPTH, CHUNK, N_FACT, TILE_W), jnp.float32),
        pltpu.VMEM((DEPTH, CHUNK, N_FACT, TILE_W), jnp.float32),
        pltpu.VMEM((BPW,), jnp.float32),
        pltpu.SemaphoreType.DMA,
        pltpu.SemaphoreType.DMA,
        pltpu.SemaphoreType.DMA,
        pltpu.SemaphoreType.DMA,
        pltpu.SemaphoreType.DMA,
        pltpu.SemaphoreType.DMA,
        pltpu.SemaphoreType.DMA,
        pltpu.SemaphoreType.DMA,
        pltpu.SemaphoreType.DMA,
        pltpu.SemaphoreType.DMA,
        pltpu.SemaphoreType.DMA,
        pltpu.SemaphoreType.DMA,
    ],
)


def _epi_body(users_h, items_h, ub_h, ib_h, bias_h, dots_h, out_h,
              idx_u, idx_i, bu_v, bi_v, bias_v, dots_v, out_v, sem_b):
    wid = lax.axis_index("s") * NC + lax.axis_index("c")
    base = wid * BPW

    pltpu.sync_copy(users_h.at[pl.ds(base, BPW)], idx_u)
    pltpu.sync_copy(items_h.at[pl.ds(base, BPW)], idx_i)
    cps = [
        pltpu.async_copy(ub_h.at[idx_u], bu_v, sem_b),
        pltpu.async_copy(ib_h.at[idx_i], bi_v, sem_b),
    ]
    pltpu.sync_copy(dots_h.at[pl.ds(base, BPW)], dots_v)
    pltpu.sync_copy(bias_h, bias_v)
    for cp in cps:
        cp.wait()
    bias_vec = bias_v[...]
    for g in range(GROUPS):
        r0 = g * L
        res = dots_v[pl.ds(r0, L)] + bu_v[pl.ds(r0, L)] + bi_v[pl.ds(r0, L)]
        res = res + bias_vec
        res = jnp.minimum(jnp.maximum(res, 1.0), 5.0)
        out_v[pl.ds(r0, L)] = res
    pltpu.sync_copy(out_v, out_h.at[pl.ds(base, BPW)])


_mf_epi = pl.kernel(
    _epi_body,
    out_type=jax.ShapeDtypeStruct((B,), jnp.float32),
    mesh=plsc.VectorSubcoreMesh(core_axis_name="c", subcore_axis_name="s"),
    compiler_params=pltpu.CompilerParams(needs_layout_passes=False),
    scratch_types=[
        pltpu.VMEM((BPW,), jnp.int32),
        pltpu.VMEM((BPW,), jnp.int32),
        pltpu.VMEM((BPW,), jnp.float32),
        pltpu.VMEM((BPW,), jnp.float32),
        pltpu.VMEM((L,), jnp.float32),
        pltpu.VMEM((BPW,), jnp.float32),
        pltpu.VMEM((BPW,), jnp.float32),
        pltpu.SemaphoreType.DMA,
    ],
)


def _pad_bias(b2d):
    flat_pad = jnp.zeros((BIAS_PAD, 1), jnp.float32)
    return jnp.concatenate([b2d, flat_pad], axis=0).reshape(-1)


def kernel(users, items, user_embeddings, item_embeddings, user_biases,
           item_biases, bias):
    u32 = users.astype(jnp.int32)
    i32 = items.astype(jnp.int32)
    uet = user_embeddings.T
    iet = item_embeddings.T
    ubp = _pad_bias(user_biases)
    ibp = _pad_bias(item_biases)
    bias16 = jnp.broadcast_to(bias.astype(jnp.float32), (L,))
    dots = _mf_main(u32, i32, uet, iet)
    return _mf_epi(u32, i32, ubp, ibp, bias16, dots)


# speedup vs baseline: 1.0105x; 1.0105x over previous
import functools

import jax
import jax.numpy as jnp
from jax import lax
from jax.experimental import pallas as pl
from jax.experimental.pallas import tpu as pltpu
from jax.experimental.pallas import tpu_sc as plsc

N_FACT = 32
N_ROWS = 1000000
B = 4096
NC = 2
NS = 16
NW = NC * NS
BPW = B // NW
L = 16
GROUPS = BPW // L
CHUNK = 2
NCH = BPW // CHUNK
TILE_W = 128
DEPTH = 6
BIAS_ROWS = 7816
BIAS_PAD = BIAS_ROWS * TILE_W - N_ROWS


def _main_body(users_h, items_h, uet_h, iet_h, out_h,
               idx_u, idx_i, ue_slabs, ie_slabs, out_v,
               sem0, sem1, sem2, sem3, sem4, sem5):
    wid = lax.axis_index("s") * NC + lax.axis_index("c")
    base = wid * BPW

    pltpu.sync_copy(users_h.at[pl.ds(base, BPW)], idx_u)
    pltpu.sync_copy(items_h.at[pl.ds(base, BPW)], idx_i)

    vu = [idx_u[pl.ds(g * L, L)] for g in range(GROUPS)]
    vi = [idx_i[pl.ds(g * L, L)] for g in range(GROUPS)]

    lane = lax.broadcasted_iota(jnp.int32, (L,), 0)
    lane_hi = lane + L
    lane_masks = [lane == j for j in range(L)]
    sems = [sem0, sem1, sem2, sem3, sem4, sem5]

    def issue(k):
        par = k % DEPTH
        cps = []
        for s in range(CHUNK):
            b = k * CHUNK + s
            g, j = b // L, b % L
            cu = vu[g][j]
            ci = vi[g][j]
            cu0 = pl.multiple_of((cu >> 7) << 7, TILE_W)
            ci0 = pl.multiple_of((ci >> 7) << 7, TILE_W)
            cps.append(pltpu.async_copy(
                uet_h.at[:, pl.ds(cu0, TILE_W)], ue_slabs.at[par, s],
                sems[par]))
            cps.append(pltpu.async_copy(
                iet_h.at[:, pl.ds(ci0, TILE_W)], ie_slabs.at[par, s],
                sems[par]))
        return cps

    acc = jnp.zeros((L,), jnp.float32)
    inflight = [issue(k) for k in range(DEPTH - 1)]
    for k in range(NCH):
        if k + DEPTH - 1 < NCH:
            inflight.append(issue(k + DEPTH - 1))
        for cp in inflight.pop(0):
            cp.wait()
        par = k % DEPTH
        for s in range(CHUNK):
            b = k * CHUNK + s
            g, j = b // L, b % L
            ju = jnp.full((L,), vu[g][j] & (TILE_W - 1), jnp.int32)
            ji = jnp.full((L,), vi[g][j] & (TILE_W - 1), jnp.int32)
            u0 = plsc.load_gather(ue_slabs.at[par, s], [lane, ju])
            u1 = plsc.load_gather(ue_slabs.at[par, s], [lane_hi, ju])
            i0 = plsc.load_gather(ie_slabs.at[par, s], [lane, ji])
            i1 = plsc.load_gather(ie_slabs.at[par, s], [lane_hi, ji])
            dot = jnp.sum(u0 * i0 + u1 * i1)
            acc = jnp.where(lane_masks[j], jnp.full((L,), dot), acc)
            if j == L - 1:
                out_v[pl.ds(g * L, L)] = acc

    pltpu.sync_copy(out_v, out_h.at[pl.ds(base, BPW)])


_mf_main = pl.kernel(
    _main_body,
    out_type=jax.ShapeDtypeStruct((B,), jnp.float32),
    mesh=plsc.VectorSubcoreMesh(core_axis_name="c", subcore_axis_name="s"),
    compiler_params=pltpu.CompilerParams(needs_layout_passes=False),
    scratch_types=[
        pltpu.VMEM((BPW,), jnp.int32),
        pltpu.VMEM((BPW,), jnp.int32),
        pltpu.VMEM((DEPTH, CHUNK, N_FACT, TILE_W), jnp.float32),
        pltpu.VMEM((DEPTH, CHUNK, N_FACT, TILE_W), jnp.float32),
        pltpu.VMEM((BPW,), jnp.float32),
        pltpu.SemaphoreType.DMA,
        pltpu.SemaphoreType.DMA,
        pltpu.SemaphoreType.DMA,
        pltpu.SemaphoreType.DMA,
        pltpu.SemaphoreType.DMA,
        pltpu.SemaphoreType.DMA,
    ],
)


def _epi_body(users_h, items_h, ub_h, ib_h, bias_h, dots_h, out_h,
              idx_u, idx_i, bu_v, bi_v, bias_v, dots_v, out_v, sem_b):
    wid = lax.axis_index("s") * NC + lax.axis_index("c")
    base = wid * BPW

    pltpu.sync_copy(users_h.at[pl.ds(base, BPW)], idx_u)
    pltpu.sync_copy(items_h.at[pl.ds(base, BPW)], idx_i)
    cps = [
        pltpu.async_copy(ub_h.at[idx_u], bu_v, sem_b),
        pltpu.async_copy(ib_h.at[idx_i], bi_v, sem_b),
    ]
    pltpu.sync_copy(dots_h.at[pl.ds(base, BPW)], dots_v)
    pltpu.sync_copy(bias_h, bias_v)
    for cp in cps:
        cp.wait()
    bias_vec = bias_v[...]
    for g in range(GROUPS):
        r0 = g * L
        res = dots_v[pl.ds(r0, L)] + bu_v[pl.ds(r0, L)] + bi_v[pl.ds(r0, L)]
        res = res + bias_vec
        res = jnp.minimum(jnp.maximum(res, 1.0), 5.0)
        out_v[pl.ds(r0, L)] = res
    pltpu.sync_copy(out_v, out_h.at[pl.ds(base, BPW)])


_mf_epi = pl.kernel(
    _epi_body,
    out_type=jax.ShapeDtypeStruct((B,), jnp.float32),
    mesh=plsc.VectorSubcoreMesh(core_axis_name="c", subcore_axis_name="s"),
    compiler_params=pltpu.CompilerParams(needs_layout_passes=False),
    scratch_types=[
        pltpu.VMEM((BPW,), jnp.int32),
        pltpu.VMEM((BPW,), jnp.int32),
        pltpu.VMEM((BPW,), jnp.float32),
        pltpu.VMEM((BPW,), jnp.float32),
        pltpu.VMEM((L,), jnp.float32),
        pltpu.VMEM((BPW,), jnp.float32),
        pltpu.VMEM((BPW,), jnp.float32),
        pltpu.SemaphoreType.DMA,
    ],
)


def _pad_bias(b2d):
    flat_pad = jnp.zeros((BIAS_PAD, 1), jnp.float32)
    return jnp.concatenate([b2d, flat_pad], axis=0).reshape(-1)


def kernel(users, items, user_embeddings, item_embeddings, user_biases,
           item_biases, bias):
    u32 = users.astype(jnp.int32)
    i32 = items.astype(jnp.int32)
    uet = user_embeddings.T
    iet = item_embeddings.T
    ubp = _pad_bias(user_biases)
    ibp = _pad_bias(item_biases)
    bias16 = jnp.broadcast_to(bias.astype(jnp.float32), (L,))
    dots = _mf_main(u32, i32, uet, iet)
    return _mf_epi(u32, i32, ubp, ibp, bias16, dots)
